# baseline (speedup 1.0000x reference)
import functools

import jax
import jax.numpy as jnp
from jax import lax
from jax.experimental import pallas as pl
from jax.experimental.pallas import tpu as pltpu
from jax.experimental.pallas import tpu_sc as plsc

_NC, _NS, _L = 2, 16, 16


def _make_sc_kernel(n, h):
    nw = _NC * _NS
    blk = 80
    assert n % blk == 0 and h % _L == 0
    nblocks = n // blk
    gmax = -(-nblocks // nw)
    nch = h // _L
    nbuf = 3

    mesh = plsc.VectorSubcoreMesh(core_axis_name="c", subcore_axis_name="s")

    @functools.partial(
        pl.kernel,
        mesh=mesh,
        out_type=jax.ShapeDtypeStruct((n, h), jnp.float32),
        scratch_types=(
            [pltpu.VMEM((3, h), jnp.float32),
             pltpu.VMEM((_L,), jnp.int32)]
            + [pltpu.VMEM((blk, h), jnp.float32)] * nbuf
            + [pltpu.SemaphoreType.DMA] * (2 * nbuf)
        ),
    )
    def sc_kernel(nodes_hbm, table_hbm, tid_hbm, out_hbm,
                  table_v, tid_v, *bufs_sems):
        wid = lax.axis_index("s") * _NC + lax.axis_index("c")

        def cond(g):
            return wid + g * nw < nblocks

        bufs = bufs_sems[:nbuf]
        sin = bufs_sems[nbuf:2 * nbuf]
        sout = bufs_sems[2 * nbuf:]

        def load(g, b):
            rs = (wid + g * nw) * blk
            return pltpu.make_async_copy(
                nodes_hbm.at[pl.ds(rs, blk)], bufs[b], sin[b])

        def store(g, b):
            rs = (wid + g * nw) * blk
            return pltpu.make_async_copy(
                bufs[b], out_hbm.at[pl.ds(rs, blk)], sout[b])

        lookahead = nbuf - 1
        for g in range(min(lookahead, gmax)):
            @pl.when(cond(g))
            def _(g=g):
                load(g, g % nbuf).start()

        pltpu.sync_copy(table_hbm, table_v)
        pltpu.sync_copy(tid_hbm, tid_v)
        tid_vec = tid_v[...]
        m0 = tid_vec == 0
        m1 = tid_vec == 1
        rowc = []
        for c in range(nch):
            r0 = table_v[0, pl.ds(c * _L, _L)]
            r1 = table_v[1, pl.ds(c * _L, _L)]
            r2 = table_v[2, pl.ds(c * _L, _L)]
            rowc.append(jnp.where(m0, r0, jnp.where(m1, r1, r2)))

        def compute(b):
            buf = bufs[b]

            def row_body(r, _):
                for c in range(nch):
                    sl = pl.ds(c * _L, _L)
                    buf[r, sl] = buf[r, sl] + rowc[c]
                return _

            lax.fori_loop(0, blk, row_body, None)

        n_super = -(-gmax // nbuf)

        def super_body(s, _):
            g0 = s * nbuf
            for b in range(nbuf):
                g = g0 + b

                @pl.when(cond(g + lookahead))
                def _(g=g, b=b):
                    @pl.when(g >= 1)
                    def _():
                        store(g - 1, (b - 1) % nbuf).wait()
                    load(g + lookahead, (b + lookahead) % nbuf).start()

                @pl.when(cond(g))
                def _(g=g, b=b):
                    load(g, b).wait()
                    compute(b)
                    store(g, b).start()
            return _

        lax.fori_loop(0, n_super, super_body, None)

        for q in range(nbuf):
            g = gmax - nbuf + q
            if g >= 0:
                store(g, g % nbuf).wait()

    return sc_kernel


def kernel(node_embeddings, task_embedding, task_id):
    n, h = node_embeddings.shape
    tid_arr = jnp.full((_L,), task_id, jnp.int32)
    sc = _make_sc_kernel(n, h)
    return sc(node_embeddings, task_embedding, tid_arr)
